# speedup vs baseline: 3.0257x; 3.0257x over previous
import functools

import jax
import jax.numpy as jnp
from jax import lax
from jax.experimental import pallas as pl
from jax.experimental.pallas import tpu as pltpu
from jax.experimental.pallas import tpu_sc as plsc

_INPUT_DIM = 2048
_D = 128
_BATCH = 4096
_SEQ = 50

_NC = 2
_NS = 16
_NW = _NC * _NS
_TOTAL = _BATCH * _SEQ
_PER_W = _TOTAL // _NW
_CHUNK = 128
_NCHUNKS = _PER_W // _CHUNK
_NBUF = 5
_NOUTER = _NCHUNKS // _NBUF - 1
_LANES = 16


def _sc_body(idx_hbm, table_hbm, out_hbm, idx_v, bufs, gsem, osem):
    wid = lax.axis_index("s") * _NC + lax.axis_index("c")
    base = wid * _PER_W

    pltpu.sync_copy(idx_hbm.at[wid], idx_v)

    def transform(r):
        for k in range(_CHUNK // _LANES):
            v = idx_v[r, pl.ds(k * _LANES, _LANES)]
            v = jnp.minimum(jnp.maximum(v, -_INPUT_DIM), _INPUT_DIM) + _INPUT_DIM
            idx_v[r, pl.ds(k * _LANES, _LANES)] = v

    def gather(c, b):
        return pltpu.make_async_copy(table_hbm.at[idx_v.at[c]], bufs.at[b],
                                     gsem.at[b])

    def out_copy(c, b):
        return pltpu.make_async_copy(
            bufs.at[b], out_hbm.at[pl.ds(base + c * _CHUNK, _CHUNK)],
            osem.at[b])

    for b in range(_NBUF):
        transform(b)
        gather(b, b).start()

    def outer(o, carry):
        c0 = o * _NBUF
        for b in range(_NBUF):
            gather(c0 + b, b).wait()
            out_copy(c0 + b, b).start()
        for b in range(_NBUF):
            transform(c0 + b + _NBUF)
            out_copy(c0 + b, b).wait()
            gather(c0 + b + _NBUF, b).start()
        return carry

    lax.fori_loop(0, _NOUTER, outer, None)

    c0 = _NOUTER * _NBUF
    for b in range(_NBUF):
        gather(c0 + b, b).wait()
        out_copy(c0 + b, b).start()
    for b in range(_NBUF):
        out_copy(c0 + b, b).wait()


@functools.partial(
    pl.kernel,
    out_type=jax.ShapeDtypeStruct((_TOTAL, _D), jnp.float32),
    mesh=plsc.VectorSubcoreMesh(core_axis_name="c", subcore_axis_name="s",
                                num_cores=_NC, num_subcores=_NS),
    scratch_types=[
        pltpu.VMEM((_NCHUNKS, _CHUNK), jnp.int32),
        pltpu.VMEM((_NBUF, _CHUNK, _D), jnp.float32),
        pltpu.SemaphoreType.DMA((_NBUF,)),
        pltpu.SemaphoreType.DMA((_NBUF,)),
    ],
)
def _position_embedding_gather(idx_hbm, table_hbm, out_hbm, idx_v, bufs,
                               gsem, osem):
    _sc_body(idx_hbm, table_hbm, out_hbm, idx_v, bufs, gsem, osem)


def kernel(inputs, embeddings):
    idx = inputs.astype(jnp.int32).reshape(_NW, _NCHUNKS, _CHUNK)
    out = _position_embedding_gather(idx, embeddings)
    return out.reshape(_BATCH, _SEQ, _D)


# speedup vs baseline: 3.0471x; 1.0071x over previous
import functools

import jax
import jax.numpy as jnp
from jax import lax
from jax.experimental import pallas as pl
from jax.experimental.pallas import tpu as pltpu
from jax.experimental.pallas import tpu_sc as plsc

_INPUT_DIM = 2048
_D = 128
_BATCH = 4096
_SEQ = 50

_NC = 2
_NS = 16
_NW = _NC * _NS
_TOTAL = _BATCH * _SEQ
_PER_W = _TOTAL // _NW
_CHUNK = 128
_NCHUNKS = _PER_W // _CHUNK
_NBUF = 5
_NOUTER = _NCHUNKS // _NBUF - 1
_LANES = 16


def _sc_body(idx_hbm, table_hbm, out_hbm, idx_v, bufs, gsem, osem):
    wid = lax.axis_index("s") * _NC + lax.axis_index("c")
    base = wid * _PER_W

    pltpu.sync_copy(idx_hbm.at[pl.ds(base, _PER_W)], idx_v)

    def transform(r):
        for k in range(_CHUNK // _LANES):
            o = r * _CHUNK + k * _LANES
            v = idx_v[pl.ds(o, _LANES)]
            v = jnp.minimum(jnp.maximum(v, -_INPUT_DIM), _INPUT_DIM) + _INPUT_DIM
            idx_v[pl.ds(o, _LANES)] = v

    def gather(c, b):
        return pltpu.make_async_copy(
            table_hbm.at[idx_v.at[pl.ds(c * _CHUNK, _CHUNK)]], bufs.at[b],
            gsem.at[b])

    def out_copy(c, b):
        return pltpu.make_async_copy(
            bufs.at[b], out_hbm.at[pl.ds(base + c * _CHUNK, _CHUNK)],
            osem.at[b])

    for b in range(_NBUF):
        transform(b)
        gather(b, b).start()

    def outer(o, carry):
        c0 = o * _NBUF
        for b in range(_NBUF):
            gather(c0 + b, b).wait()
            out_copy(c0 + b, b).start()
        for b in range(_NBUF):
            transform(c0 + b + _NBUF)
            out_copy(c0 + b, b).wait()
            gather(c0 + b + _NBUF, b).start()
        return carry

    lax.fori_loop(0, _NOUTER, outer, None)

    c0 = _NOUTER * _NBUF
    for b in range(_NBUF):
        gather(c0 + b, b).wait()
        out_copy(c0 + b, b).start()
    for b in range(_NBUF):
        out_copy(c0 + b, b).wait()


@functools.partial(
    pl.kernel,
    out_type=jax.ShapeDtypeStruct((_TOTAL, _D), jnp.float32),
    mesh=plsc.VectorSubcoreMesh(core_axis_name="c", subcore_axis_name="s",
                                num_cores=_NC, num_subcores=_NS),
    compiler_params=pltpu.CompilerParams(use_tc_tiling_on_sc=True),
    scratch_types=[
        pltpu.VMEM((_PER_W,), jnp.int32),
        pltpu.VMEM((_NBUF, _CHUNK, _D), jnp.float32),
        pltpu.SemaphoreType.DMA((_NBUF,)),
        pltpu.SemaphoreType.DMA((_NBUF,)),
    ],
)
def _position_embedding_gather(idx_hbm, table_hbm, out_hbm, idx_v, bufs,
                               gsem, osem):
    _sc_body(idx_hbm, table_hbm, out_hbm, idx_v, bufs, gsem, osem)


def kernel(inputs, embeddings):
    idx = inputs.astype(jnp.int32).reshape(_TOTAL)
    out = _position_embedding_gather(idx, embeddings)
    return out.reshape(_BATCH, _SEQ, _D)


# speedup vs baseline: 8.1866x; 2.6867x over previous
import functools

import jax
import jax.numpy as jnp
from jax import lax
from jax.experimental import pallas as pl
from jax.experimental.pallas import tpu as pltpu
from jax.experimental.pallas import tpu_sc as plsc

_INPUT_DIM = 2048
_D = 128
_BATCH = 4096
_SEQ = 50

_NC = 2
_NS = 16
_NW = _NC * _NS
_BPW = _BATCH // _NW
_NCHUNKS = _SEQ
_NBUF = 5
_NOUTER = _NCHUNKS // _NBUF - 1
_LANES = 16


def _sc_body(idx_hbm, table_hbm, out_hbm, idx_v, bufs, gsem, osem):
    wid = lax.axis_index("s") * _NC + lax.axis_index("c")
    base = wid * _BPW

    pltpu.sync_copy(idx_hbm.at[:, pl.ds(base, _BPW)], idx_v)

    def transform(r):
        for k in range(_BPW // _LANES):
            v = idx_v[r, pl.ds(k * _LANES, _LANES)]
            v = jnp.minimum(jnp.maximum(v, -_INPUT_DIM), _INPUT_DIM) + _INPUT_DIM
            idx_v[r, pl.ds(k * _LANES, _LANES)] = v

    def gather(c, b):
        return pltpu.make_async_copy(table_hbm.at[idx_v.at[c]], bufs.at[b],
                                     gsem.at[b])

    def out_copy(c, b):
        return pltpu.make_async_copy(bufs.at[b],
                                     out_hbm.at[c, pl.ds(base, _BPW)],
                                     osem.at[b])

    for b in range(_NBUF):
        transform(b)
        gather(b, b).start()

    def outer(o, carry):
        c0 = o * _NBUF
        for b in range(_NBUF):
            gather(c0 + b, b).wait()
            out_copy(c0 + b, b).start()
        for b in range(_NBUF):
            transform(c0 + b + _NBUF)
            out_copy(c0 + b, b).wait()
            gather(c0 + b + _NBUF, b).start()
        return carry

    lax.fori_loop(0, _NOUTER, outer, None)

    c0 = _NOUTER * _NBUF
    for b in range(_NBUF):
        gather(c0 + b, b).wait()
        out_copy(c0 + b, b).start()
    for b in range(_NBUF):
        out_copy(c0 + b, b).wait()


@functools.partial(
    pl.kernel,
    out_type=jax.ShapeDtypeStruct((_SEQ, _BATCH, _D), jnp.float32),
    mesh=plsc.VectorSubcoreMesh(core_axis_name="c", subcore_axis_name="s",
                                num_cores=_NC, num_subcores=_NS),
    scratch_types=[
        pltpu.VMEM((_SEQ, _BPW), jnp.int32),
        pltpu.VMEM((_NBUF, _BPW, _D), jnp.float32),
        pltpu.SemaphoreType.DMA((_NBUF,)),
        pltpu.SemaphoreType.DMA((_NBUF,)),
    ],
)
def _position_embedding_gather(idx_hbm, table_hbm, out_hbm, idx_v, bufs,
                               gsem, osem):
    _sc_body(idx_hbm, table_hbm, out_hbm, idx_v, bufs, gsem, osem)


def kernel(inputs, embeddings):
    idx = inputs.astype(jnp.int32).T
    out = _position_embedding_gather(idx, embeddings)
    return jnp.transpose(out, (1, 0, 2))


# speedup vs baseline: 14.7876x; 1.8063x over previous
import functools

import jax
import jax.numpy as jnp
from jax import lax
from jax.experimental import pallas as pl
from jax.experimental.pallas import tpu as pltpu
from jax.experimental.pallas import tpu_sc as plsc

_INPUT_DIM = 2048
_D = 128
_BATCH = 4096
_SEQ = 50

_NC = 2
_NS = 16
_NW = _NC * _NS
_BPW = _BATCH // _NW
_NCHUNKS = _SEQ
_NBUF = 5
_NOUTER = _NCHUNKS // _NBUF - 1
_LANES = 16


def _sc_body(idx_hbm, table_hbm, out_hbm, idx_v, bufs, table_sh, gsem, osem):
    sid = lax.axis_index("s")
    wid = sid * _NC + lax.axis_index("c")
    base = wid * _BPW

    rows = (_INPUT_DIM * 2 + 1) // _NS
    pltpu.sync_copy(table_hbm.at[pl.ds(sid * rows, rows)],
                    table_sh.at[pl.ds(sid * rows, rows)])

    @pl.when(sid == 0)
    def _():
        pltpu.sync_copy(table_hbm.at[pl.ds(_NS * rows, 1)],
                        table_sh.at[pl.ds(_NS * rows, 1)])

    pltpu.sync_copy(idx_hbm.at[:, pl.ds(base, _BPW)], idx_v)
    plsc.subcore_barrier()

    def transform(r):
        for k in range(_BPW // _LANES):
            v = idx_v[r, pl.ds(k * _LANES, _LANES)]
            v = jnp.minimum(jnp.maximum(v, -_INPUT_DIM), _INPUT_DIM) + _INPUT_DIM
            idx_v[r, pl.ds(k * _LANES, _LANES)] = v

    def gather(c, b):
        return pltpu.make_async_copy(table_sh.at[idx_v.at[c]], bufs.at[b],
                                     gsem.at[b])

    def out_copy(c, b):
        return pltpu.make_async_copy(bufs.at[b],
                                     out_hbm.at[c, pl.ds(base, _BPW)],
                                     osem.at[b])

    for b in range(_NBUF):
        transform(b)
        gather(b, b).start()

    def outer(o, carry):
        c0 = o * _NBUF
        for b in range(_NBUF):
            gather(c0 + b, b).wait()
            out_copy(c0 + b, b).start()
        for b in range(_NBUF):
            transform(c0 + b + _NBUF)
            out_copy(c0 + b, b).wait()
            gather(c0 + b + _NBUF, b).start()
        return carry

    lax.fori_loop(0, _NOUTER, outer, None)

    c0 = _NOUTER * _NBUF
    for b in range(_NBUF):
        gather(c0 + b, b).wait()
        out_copy(c0 + b, b).start()
    for b in range(_NBUF):
        out_copy(c0 + b, b).wait()


@functools.partial(
    pl.kernel,
    out_type=jax.ShapeDtypeStruct((_SEQ, _BATCH, _D), jnp.float32),
    mesh=plsc.VectorSubcoreMesh(core_axis_name="c", subcore_axis_name="s",
                                num_cores=_NC, num_subcores=_NS),
    scratch_types=[
        pltpu.VMEM((_SEQ, _BPW), jnp.int32),
        pltpu.VMEM((_NBUF, _BPW, _D), jnp.float32),
        pltpu.VMEM_SHARED((_INPUT_DIM * 2 + 1, _D), jnp.float32),
        pltpu.SemaphoreType.DMA((_NBUF,)),
        pltpu.SemaphoreType.DMA((_NBUF,)),
    ],
)
def _position_embedding_gather(idx_hbm, table_hbm, out_hbm, idx_v, bufs,
                               table_sh, gsem, osem):
    _sc_body(idx_hbm, table_hbm, out_hbm, idx_v, bufs, table_sh, gsem, osem)


def kernel(inputs, embeddings):
    idx = inputs.astype(jnp.int32).T
    out = _position_embedding_gather(idx, embeddings)
    return jnp.transpose(out, (1, 0, 2))


# speedup vs baseline: 15.1070x; 1.0216x over previous
import functools

import jax
import jax.numpy as jnp
from jax import lax
from jax.experimental import pallas as pl
from jax.experimental.pallas import tpu as pltpu
from jax.experimental.pallas import tpu_sc as plsc

_INPUT_DIM = 2048
_D = 128
_BATCH = 4096
_SEQ = 50

_NC = 2
_NS = 16
_NW = _NC * _NS
_BPW = _BATCH // _NW
_NCHUNKS = _SEQ
_NBUF = 5
_NOUTER = _NCHUNKS // _NBUF - 1
_LANES = 16


def _sc_body(idx_hbm, table_hbm, out_hbm, idx_v, bufs, table_sh, gsem, osem):
    sid = lax.axis_index("s")
    wid = sid * _NC + lax.axis_index("c")
    base = wid * _BPW

    rows = (_INPUT_DIM * 2 + 1) // _NS
    tcp = pltpu.make_async_copy(table_hbm.at[pl.ds(sid * rows, rows)],
                                table_sh.at[pl.ds(sid * rows, rows)],
                                gsem.at[0])
    tcp.start()
    icp = pltpu.make_async_copy(idx_hbm.at[:, pl.ds(base, _BPW)], idx_v,
                                gsem.at[1])
    icp.start()

    @pl.when(sid == 0)
    def _():
        pltpu.sync_copy(table_hbm.at[pl.ds(_NS * rows, 1)],
                        table_sh.at[pl.ds(_NS * rows, 1)])

    tcp.wait()
    icp.wait()
    plsc.subcore_barrier()

    def transform(r):
        for k in range(_BPW // _LANES):
            v = idx_v[r, pl.ds(k * _LANES, _LANES)]
            v = jnp.minimum(jnp.maximum(v, -_INPUT_DIM), _INPUT_DIM) + _INPUT_DIM
            idx_v[r, pl.ds(k * _LANES, _LANES)] = v

    def gather(c, b):
        return pltpu.make_async_copy(table_sh.at[idx_v.at[c]], bufs.at[b],
                                     gsem.at[b])

    def out_copy(c, b):
        return pltpu.make_async_copy(bufs.at[b],
                                     out_hbm.at[c, pl.ds(base, _BPW)],
                                     osem.at[b])

    for b in range(_NBUF):
        transform(b)
        gather(b, b).start()

    def outer(o, carry):
        c0 = o * _NBUF
        for b in range(_NBUF):
            gather(c0 + b, b).wait()
            out_copy(c0 + b, b).start()
        for b in range(_NBUF):
            transform(c0 + b + _NBUF)
            out_copy(c0 + b, b).wait()
            gather(c0 + b + _NBUF, b).start()
        return carry

    lax.fori_loop(0, _NOUTER, outer, None)

    c0 = _NOUTER * _NBUF
    for b in range(_NBUF):
        gather(c0 + b, b).wait()
        out_copy(c0 + b, b).start()
    for b in range(_NBUF):
        out_copy(c0 + b, b).wait()


@functools.partial(
    pl.kernel,
    out_type=jax.ShapeDtypeStruct((_SEQ, _BATCH, _D), jnp.float32),
    mesh=plsc.VectorSubcoreMesh(core_axis_name="c", subcore_axis_name="s",
                                num_cores=_NC, num_subcores=_NS),
    scratch_types=[
        pltpu.VMEM((_SEQ, _BPW), jnp.int32),
        pltpu.VMEM((_NBUF, _BPW, _D), jnp.float32),
        pltpu.VMEM_SHARED((_INPUT_DIM * 2 + 1, _D), jnp.float32),
        pltpu.SemaphoreType.DMA((_NBUF,)),
        pltpu.SemaphoreType.DMA((_NBUF,)),
    ],
)
def _position_embedding_gather(idx_hbm, table_hbm, out_hbm, idx_v, bufs,
                               table_sh, gsem, osem):
    _sc_body(idx_hbm, table_hbm, out_hbm, idx_v, bufs, table_sh, gsem, osem)


def kernel(inputs, embeddings):
    idx = inputs.astype(jnp.int32).T
    out = _position_embedding_gather(idx, embeddings)
    return jnp.transpose(out, (1, 0, 2))
